# baseline (speedup 1.0000x reference)
import functools

import jax
import jax.numpy as jnp
from jax import lax
from jax.experimental import pallas as pl
from jax.experimental.pallas import tpu as pltpu
from jax.experimental.pallas import tpu_sc as plsc

NC = 2
NS = 16
L = 16
NW = NC * NS

D = 64
CHUNK = 128
NBUF = 4


def kernel(t, emb):
    A, B, Dp = t.shape
    assert Dp == D
    V, De = emb.shape
    assert De == D
    R = A * B
    assert R % NW == 0
    rows_per_w = R // NW
    assert rows_per_w % (CHUNK * NBUF) == 0
    n_chunks = rows_per_w // CHUNK
    n_groups = n_chunks // NBUF

    t2 = t.reshape(R, D)
    mesh = plsc.VectorSubcoreMesh(core_axis_name="c", subcore_axis_name="s")

    @functools.partial(
        pl.kernel,
        mesh=mesh,
        compiler_params=pltpu.CompilerParams(
            use_tc_tiling_on_sc=False, needs_layout_passes=False
        ),
        out_type=jax.ShapeDtypeStruct((R, 2 * D), jnp.float32),
        scratch_types=[
            pltpu.VMEM((NBUF, CHUNK, 2 * D), jnp.float32),
            pltpu.VMEM((NBUF, CHUNK, D), jnp.float32),
            pltpu.VMEM((NBUF, CHUNK), jnp.int32),
            pltpu.SemaphoreType.DMA,
            pltpu.SemaphoreType.DMA,
            pltpu.SemaphoreType.DMA,
            pltpu.SemaphoreType.DMA,
            pltpu.SemaphoreType.DMA,
            pltpu.SemaphoreType.DMA,
            pltpu.SemaphoreType.DMA,
            pltpu.SemaphoreType.DMA,
            pltpu.SemaphoreType.DMA,
            pltpu.SemaphoreType.DMA,
            pltpu.SemaphoreType.DMA,
            pltpu.SemaphoreType.DMA,
        ],
    )
    def body(t_hbm, emb_hbm, out_hbm, stage, ebuf, iv,
             is0, is1, is2, is3, gs0, gs1, gs2, gs3, os0, os1, os2, os3):
        isem = [is0, is1, is2, is3]
        gsem = [gs0, gs1, gs2, gs3]
        osem = [os0, os1, os2, os3]

        wid = lax.axis_index("s") * NC + lax.axis_index("c")
        wbase = wid * rows_per_w

        lane = lax.iota(jnp.int32, L)
        col2 = jnp.full((L,), 2, jnp.int32)
        vmax = jnp.full((L,), V - 1, jnp.int32)
        zero = jnp.zeros((L,), jnp.int32)

        def in_copy(ci, b):
            return pltpu.make_async_copy(
                t_hbm.at[pl.ds(wbase + ci * CHUNK, CHUNK)],
                stage.at[b, :, pl.ds(0, D)],
                isem[b],
            )

        def gather_copy(b):
            return pltpu.make_async_copy(
                emb_hbm.at[iv.at[b]], ebuf.at[b], gsem[b]
            )

        def out_copy(ci, b):
            return pltpu.make_async_copy(
                stage.at[b], out_hbm.at[pl.ds(wbase + ci * CHUNK, CHUNK)],
                osem[b],
            )

        in_copy(0, 0).start()

        def group_body(g, carry):
            for b in range(NBUF):
                ci = g * NBUF + b

                in_copy(ci, b).wait()

                def idx_body(j, carry2):
                    rows = lane + j * L
                    vals = plsc.load_gather(stage.at[b], [rows, col2])
                    idx = jnp.minimum(
                        jnp.maximum(vals.astype(jnp.int32), zero), vmax
                    )
                    iv[b, pl.ds(j * L, L)] = idx
                    return carry2

                lax.fori_loop(0, CHUNK // L, idx_body, 0)

                gather_copy(b).start()

                pb = (b - 1) % NBUF
                def service(pci, sb):
                    gather_copy(sb).wait()

                    def copy_body(r, c2):
                        for q in range(D // L):
                            stage[sb, r, pl.ds(D + q * L, L)] = (
                                ebuf[sb, r, pl.ds(q * L, L)]
                            )
                        return c2

                    lax.fori_loop(0, CHUNK, copy_body, 0)
                    out_copy(pci, sb).start()

                if b == 0:
                    @pl.when(g >= 1)
                    def _():
                        service(ci - 1, pb)
                else:
                    service(ci - 1, pb)

                nb = (b + 1) % NBUF
                if b < NBUF - 1:
                    @pl.when(g >= 1)
                    def _():
                        out_copy(ci - 3, nb).wait()

                    @pl.when(ci + 1 < n_chunks)
                    def _():
                        in_copy(ci + 1, nb).start()
                else:
                    out_copy(ci - 3, nb).wait()

                    @pl.when(ci + 1 < n_chunks)
                    def _():
                        in_copy(ci + 1, nb).start()
            return carry

        lax.fori_loop(0, n_groups, group_body, 0)

        last = n_chunks - 1
        bl = last % NBUF
        gather_copy(bl).wait()

        def fin_copy_body(r, c2):
            for q in range(D // L):
                stage[bl, r, pl.ds(D + q * L, L)] = ebuf[bl, r, pl.ds(q * L, L)]
            return c2

        lax.fori_loop(0, CHUNK, fin_copy_body, 0)
        out_copy(last, bl).start()
        out_copy(last - 2, (bl - 2) % NBUF).wait()
        out_copy(last - 1, (bl - 1) % NBUF).wait()
        out_copy(last, bl).wait()

    out2 = body(t2, emb)
    return out2.reshape(A, B, 2 * D)
